# Initial kernel scaffold; baseline (speedup 1.0000x reference)
import jax
import jax.numpy as jnp
from jax.experimental import pallas as pl


def kernel(feature_nchw, w_conv, b_conv):
    raise NotImplementedError("write your pallas kernel here")


# speedup vs baseline: 2.4684x; 2.4684x over previous
import functools

import jax
import jax.numpy as jnp
from jax.experimental import pallas as pl
from jax.experimental.pallas import tpu as pltpu


def _shift_offsets(k):
    half = k // 2
    shifts = [(i, i) for i in range(k)]
    shifts += [(i, half) for i in range(k)]
    shifts += [(i, k - 1 - i) for i in range(k)]
    shifts += [(half, i) for i in range(k)]
    return tuple(shifts)


def _escn_body(x_ref, wx_ref, w2_ref, o_ref, pad_ref, *, C, HW, W, PAD, SHIFTS):
    x = x_ref[0]

    ssq = jnp.sum(x * x, axis=0, keepdims=True)
    xn = x * jax.lax.rsqrt(jnp.maximum(ssq, 1e-24))

    padw = pad_ref.shape[1] - HW
    pad_ref[:, :HW] = xn
    pad_ref[:, HW:] = jnp.zeros((C, padw), jnp.float32)

    wcol = jax.lax.broadcasted_iota(jnp.int32, (1, HW), 1) & (W - 1)
    col_masks = {}

    def col_mask(d):
        if d not in col_masks:
            if d > 0:
                col_masks[d] = (wcol <= (W - 1 - d)).astype(jnp.float32)
            else:
                col_masks[d] = (wcol >= -d).astype(jnp.float32)
        return col_masks[d]

    groups = C // 8

    def pos_map(delta, dcol):
        win = pad_ref[:, delta:delta + HW]
        prod = win * xn
        p8 = jnp.sum(prod.reshape(groups, 8, HW), axis=0)
        if dcol != 0:
            p8 = p8 * col_mask(dcol)
        return p8

    def neg_map(p_pos, delta):
        return jnp.concatenate(
            [jnp.zeros((8, delta), jnp.float32), p_pos[:, :HW - delta]],
            axis=1)

    cache = {}
    parts = []
    for (r, c) in SHIFTS:
        if (r, c) not in cache:
            delta = (r - PAD) * W + (c - PAD)
            if delta >= 0:
                cache[(r, c)] = pos_map(delta, c - PAD)
            else:
                mirror = (2 * PAD - r, 2 * PAD - c)
                if mirror not in cache:
                    cache[mirror] = pos_map(-delta, PAD - c)
                cache[(r, c)] = neg_map(cache[mirror], -delta)
        parts.append(cache[(r, c)])

    sub = jax.lax.broadcasted_iota(jnp.int32, (8, HW), 0)
    parts.append(jnp.where(sub == 0, 1.0, 0.0))

    pp = jnp.concatenate(parts, axis=0)
    acc = jnp.dot(wx_ref[...], x, preferred_element_type=jnp.float32)
    acc = acc + jnp.dot(w2_ref[...], pp, preferred_element_type=jnp.float32)
    o_ref[0] = acc.astype(o_ref.dtype)


def kernel(feature_nchw, w_conv, b_conv):
    b, c, h, w = feature_nchw.shape
    k = 7
    pad = k // 2
    cout = w_conv.shape[0]
    assert w_conv.shape[1] == c + 4 * k
    hw = h * w
    shifts = _shift_offsets(k)

    xflat = feature_nchw.reshape(b, c, hw)
    wx = w_conv[:, :c]
    w2 = jnp.concatenate(
        [jnp.repeat(w_conv[:, c:], 8, axis=1),
         b_conv.reshape(cout, 1),
         jnp.zeros((cout, 7), w_conv.dtype)], axis=1)

    body = functools.partial(_escn_body, C=c, HW=hw, W=w, PAD=pad,
                             SHIFTS=shifts)
    padw = ((pad * w + pad + 127) // 128) * 128

    out = pl.pallas_call(
        body,
        out_shape=jax.ShapeDtypeStruct((b, cout, hw), feature_nchw.dtype),
        grid=(b,),
        in_specs=[
            pl.BlockSpec((1, c, hw), lambda i: (i, 0, 0)),
            pl.BlockSpec((cout, c), lambda i: (0, 0)),
            pl.BlockSpec((cout, 4 * k * 8 + 8), lambda i: (0, 0)),
        ],
        out_specs=pl.BlockSpec((1, cout, hw), lambda i: (i, 0, 0)),
        scratch_shapes=[pltpu.VMEM((c, hw + padw), jnp.float32)],
        compiler_params=pltpu.CompilerParams(
            dimension_semantics=("parallel",)),
    )(xflat, wx, w2)

    return out.reshape(b, cout, h, w)


# speedup vs baseline: 2.5875x; 1.0483x over previous
import functools

import jax
import jax.numpy as jnp
from jax.experimental import pallas as pl
from jax.experimental.pallas import tpu as pltpu


def _shift_offsets(k):
    half = k // 2
    shifts = [(i, i) for i in range(k)]
    shifts += [(i, half) for i in range(k)]
    shifts += [(i, k - 1 - i) for i in range(k)]
    shifts += [(half, i) for i in range(k)]
    return tuple(shifts)


def _escn_body(x_ref, wx_ref, w2_ref, o_ref, pad_ref, *,
               C, HW, W, PAD, SHIFTS, CHUNK):
    groups = C // 8

    x = x_ref[0]
    ssq = jnp.sum(x * x, axis=0, keepdims=True)
    pad_ref[:, :HW] = x * jax.lax.rsqrt(jnp.maximum(ssq, 1e-24))
    pad_ref[:, HW:] = jnp.zeros((C, pad_ref.shape[1] - HW), jnp.float32)

    pos_keys = []
    for (r, c) in SHIFTS:
        delta = (r - PAD) * W + (c - PAD)
        key = (delta, c - PAD) if delta >= 0 else (-delta, PAD - c)
        if key not in pos_keys:
            pos_keys.append(key)

    wcol = jax.lax.broadcasted_iota(jnp.int32, (1, HW), 1) & (W - 1)
    col_masks = {}
    for (_, d) in pos_keys:
        if d > 0 and d not in col_masks:
            col_masks[d] = (wcol <= (W - 1 - d)).astype(jnp.float32)
        elif d < 0 and d not in col_masks:
            col_masks[d] = (wcol >= -d).astype(jnp.float32)

    sub = jax.lax.broadcasted_iota(jnp.int32, (8, CHUNK), 0)
    ones_tile = jnp.where(sub == 0, 1.0, 0.0)

    zeros8 = jnp.zeros((8, CHUNK), jnp.float32)
    prev = {key: zeros8 for key in pos_keys}

    for c0 in range(0, HW, CHUNK):
        xc = pad_ref[:, c0:c0 + CHUNK]
        cur = {}
        for (delta, d) in pos_keys:
            win = pad_ref[:, c0 + delta:c0 + delta + CHUNK]
            prod = win * xc
            p8 = jnp.sum(prod.reshape(groups, 8, CHUNK), axis=0)
            if d != 0:
                p8 = p8 * col_masks[d][:, c0:c0 + CHUNK]
            cur[(delta, d)] = p8

        parts = []
        for (r, c) in SHIFTS:
            delta = (r - PAD) * W + (c - PAD)
            if delta >= 0:
                parts.append(cur[(delta, c - PAD)])
            else:
                key = (-delta, PAD - c)
                parts.append(jnp.concatenate(
                    [prev[key][:, CHUNK + delta:], cur[key][:, :CHUNK + delta]],
                    axis=1))
        parts.append(ones_tile)
        pp = jnp.concatenate(parts, axis=0)

        xraw = x_ref[0, :, pl.ds(c0, CHUNK)]
        acc = jnp.dot(wx_ref[...], xraw, preferred_element_type=jnp.float32)
        acc = acc + jnp.dot(w2_ref[...], pp, preferred_element_type=jnp.float32)
        o_ref[0, :, pl.ds(c0, CHUNK)] = acc.astype(o_ref.dtype)
        prev = cur


def kernel(feature_nchw, w_conv, b_conv):
    b, c, h, w = feature_nchw.shape
    k = 7
    pad = k // 2
    cout = w_conv.shape[0]
    assert w_conv.shape[1] == c + 4 * k
    hw = h * w
    shifts = _shift_offsets(k)

    xflat = feature_nchw.reshape(b, c, hw)
    wx = w_conv[:, :c]
    w2 = jnp.concatenate(
        [jnp.repeat(w_conv[:, c:], 8, axis=1),
         b_conv.reshape(cout, 1),
         jnp.zeros((cout, 7), w_conv.dtype)], axis=1)

    body = functools.partial(_escn_body, C=c, HW=hw, W=w, PAD=pad,
                             SHIFTS=shifts, CHUNK=1024)
    padw = ((pad * w + pad + 127) // 128) * 128

    out = pl.pallas_call(
        body,
        out_shape=jax.ShapeDtypeStruct((b, cout, hw), feature_nchw.dtype),
        grid=(b,),
        in_specs=[
            pl.BlockSpec((1, c, hw), lambda i: (i, 0, 0)),
            pl.BlockSpec((cout, c), lambda i: (0, 0)),
            pl.BlockSpec((cout, 4 * k * 8 + 8), lambda i: (0, 0)),
        ],
        out_specs=pl.BlockSpec((1, cout, hw), lambda i: (i, 0, 0)),
        scratch_shapes=[
            pltpu.VMEM((c, hw + padw), jnp.float32),
        ],
        compiler_params=pltpu.CompilerParams(
            dimension_semantics=("parallel",)),
    )(xflat, wx, w2)

    return out.reshape(b, cout, h, w)


# speedup vs baseline: 3.0029x; 1.1605x over previous
import functools

import jax
import jax.numpy as jnp
from jax.experimental import pallas as pl
from jax.experimental.pallas import tpu as pltpu


def _shift_offsets(k):
    half = k // 2
    shifts = [(i, i) for i in range(k)]
    shifts += [(i, half) for i in range(k)]
    shifts += [(i, k - 1 - i) for i in range(k)]
    shifts += [(half, i) for i in range(k)]
    return tuple(shifts)


def _escn_body(x_ref, wx_ref, w2_ref, o_ref, pad_ref, *,
               C, HW, W, PAD, SHIFTS, CHUNK):
    groups = C // 8

    def _norm(x):
        ssq = jnp.sum(x * x, axis=0, keepdims=True)
        return x * jax.lax.rsqrt(jnp.maximum(ssq, 1e-24))

    xn_pair = pltpu.pack_elementwise(
        [_norm(x_ref[0]), _norm(x_ref[1])], packed_dtype=jnp.bfloat16)
    pad_ref[:, :HW] = xn_pair
    pad_ref[:, HW:] = jnp.zeros((C, pad_ref.shape[1] - HW), jnp.int32)

    pos_keys = []
    for (r, c) in SHIFTS:
        delta = (r - PAD) * W + (c - PAD)
        key = (delta, c - PAD) if delta >= 0 else (-delta, PAD - c)
        if key not in pos_keys:
            pos_keys.append(key)

    wcol = jax.lax.broadcasted_iota(jnp.int32, (1, HW), 1) & (W - 1)
    col_masks = {}
    for (_, d) in pos_keys:
        if d > 0 and d not in col_masks:
            col_masks[d] = (wcol <= (W - 1 - d)).astype(jnp.float32)
        elif d < 0 and d not in col_masks:
            col_masks[d] = (wcol >= -d).astype(jnp.float32)

    sub = jax.lax.broadcasted_iota(jnp.int32, (8, CHUNK), 0)
    ones_tile = jnp.where(sub == 0, 1.0, 0.0)

    zeros8 = jnp.zeros((8, CHUNK), jnp.float32)
    prev = {key: (zeros8, zeros8) for key in pos_keys}

    for c0 in range(0, HW, CHUNK):
        xc_bf = pltpu.bitcast(pad_ref[:, c0:c0 + CHUNK], jnp.bfloat16)
        cur = {}
        for (delta, d) in pos_keys:
            win = pad_ref[:, c0 + delta:c0 + delta + CHUNK]
            prod = pltpu.bitcast(win, jnp.bfloat16) * xc_bf
            p16 = jnp.sum(prod.reshape(groups, 16, CHUNK), axis=0)
            p16_i = pltpu.bitcast(p16, jnp.int32)
            ps = [pltpu.unpack_elementwise(
                      p16_i, index=i, packed_dtype=jnp.bfloat16,
                      unpacked_dtype=jnp.float32) for i in (0, 1)]
            if d != 0:
                m = col_masks[d][:, c0:c0 + CHUNK]
                ps = [p * m for p in ps]
            cur[(delta, d)] = ps

        outs = []
        for bi in range(2):
            parts = []
            for (r, c) in SHIFTS:
                delta = (r - PAD) * W + (c - PAD)
                if delta >= 0:
                    parts.append(cur[(delta, c - PAD)][bi])
                else:
                    key = (-delta, PAD - c)
                    parts.append(jnp.concatenate(
                        [prev[key][bi][:, CHUNK + delta:],
                         cur[key][bi][:, :CHUNK + delta]], axis=1))
            parts.append(ones_tile)
            pp = jnp.concatenate(parts, axis=0)
            xraw = x_ref[bi, :, pl.ds(c0, CHUNK)]
            acc = jnp.dot(wx_ref[...], xraw, preferred_element_type=jnp.float32)
            acc = acc + jnp.dot(w2_ref[...], pp,
                                preferred_element_type=jnp.float32)
            outs.append(acc)
        o_ref[0, :, pl.ds(c0, CHUNK)] = outs[0].astype(o_ref.dtype)
        o_ref[1, :, pl.ds(c0, CHUNK)] = outs[1].astype(o_ref.dtype)
        prev = cur


def kernel(feature_nchw, w_conv, b_conv):
    b, c, h, w = feature_nchw.shape
    k = 7
    pad = k // 2
    cout = w_conv.shape[0]
    assert w_conv.shape[1] == c + 4 * k
    hw = h * w
    shifts = _shift_offsets(k)

    xflat = feature_nchw.reshape(b, c, hw)
    wx = w_conv[:, :c]
    w2 = jnp.concatenate(
        [jnp.repeat(w_conv[:, c:], 8, axis=1),
         b_conv.reshape(cout, 1),
         jnp.zeros((cout, 7), w_conv.dtype)], axis=1)

    body = functools.partial(_escn_body, C=c, HW=hw, W=w, PAD=pad,
                             SHIFTS=shifts, CHUNK=1024)
    padw = ((pad * w + pad + 127) // 128) * 128

    out = pl.pallas_call(
        body,
        out_shape=jax.ShapeDtypeStruct((b, cout, hw), feature_nchw.dtype),
        grid=(b // 2,),
        in_specs=[
            pl.BlockSpec((2, c, hw), lambda i: (i, 0, 0)),
            pl.BlockSpec((cout, c), lambda i: (0, 0)),
            pl.BlockSpec((cout, 4 * k * 8 + 8), lambda i: (0, 0)),
        ],
        out_specs=pl.BlockSpec((2, cout, hw), lambda i: (i, 0, 0)),
        scratch_shapes=[
            pltpu.VMEM((c, hw + padw), jnp.int32),
        ],
        compiler_params=pltpu.CompilerParams(
            dimension_semantics=("parallel",)),
    )(xflat, wx, w2)

    return out.reshape(b, cout, h, w)


# speedup vs baseline: 3.6421x; 1.2129x over previous
import functools

import jax
import jax.numpy as jnp
from jax.experimental import pallas as pl
from jax.experimental.pallas import tpu as pltpu


def _shift_offsets(k):
    half = k // 2
    shifts = [(i, i) for i in range(k)]
    shifts += [(i, half) for i in range(k)]
    shifts += [(i, k - 1 - i) for i in range(k)]
    shifts += [(half, i) for i in range(k)]
    return tuple(shifts)


def _escn_body(x_ref, wx_ref, w2_ref, o_ref, pad_ref, *,
               C, HW, W, PAD, SHIFTS, CHUNK):
    groups = C // 8

    def _norm(x):
        ssq = jnp.sum(x * x, axis=0, keepdims=True)
        return x * jax.lax.rsqrt(jnp.maximum(ssq, 1e-24))

    xn_pair = pltpu.pack_elementwise(
        [_norm(x_ref[0]), _norm(x_ref[1])], packed_dtype=jnp.bfloat16)
    pad_ref[:, :HW] = xn_pair
    pad_ref[:, HW:] = jnp.zeros((C, pad_ref.shape[1] - HW), jnp.int32)

    pos_keys = []
    for (r, c) in SHIFTS:
        delta = (r - PAD) * W + (c - PAD)
        key = (delta, c - PAD) if delta >= 0 else (-delta, PAD - c)
        if key not in pos_keys:
            pos_keys.append(key)

    wcol = jax.lax.broadcasted_iota(jnp.int32, (1, HW), 1) & (W - 1)
    col_masks = {}
    for (_, d) in pos_keys:
        if d > 0 and d not in col_masks:
            col_masks[d] = (wcol <= (W - 1 - d)).astype(jnp.float32)
        elif d < 0 and d not in col_masks:
            col_masks[d] = (wcol >= -d).astype(jnp.float32)

    sub = jax.lax.broadcasted_iota(jnp.int32, (8, CHUNK), 0)
    ones_tile = jnp.where(sub == 0, 1.0, 0.0)

    zeros8 = jnp.zeros((8, CHUNK), jnp.float32)
    prev = {key: (zeros8, zeros8) for key in pos_keys}

    for c0 in range(0, HW, CHUNK):
        xc_bf = pltpu.bitcast(pad_ref[:, c0:c0 + CHUNK], jnp.bfloat16)
        cur = {}
        for (delta, d) in pos_keys:
            win = pad_ref[:, c0 + delta:c0 + delta + CHUNK]
            prod = pltpu.bitcast(win, jnp.bfloat16) * xc_bf
            p16 = jnp.sum(prod.reshape(groups, 16, CHUNK), axis=0)
            p16_i = pltpu.bitcast(p16, jnp.int32)
            ps = [pltpu.unpack_elementwise(
                      p16_i, index=i, packed_dtype=jnp.bfloat16,
                      unpacked_dtype=jnp.float32) for i in (0, 1)]
            if d != 0:
                m = col_masks[d][:, c0:c0 + CHUNK]
                ps = [p * m for p in ps]
            cur[(delta, d)] = ps

        outs = []
        for bi in range(2):
            parts = []
            for (r, c) in SHIFTS:
                delta = (r - PAD) * W + (c - PAD)
                if delta >= 0:
                    parts.append(cur[(delta, c - PAD)][bi])
                else:
                    key = (-delta, PAD - c)
                    parts.append(jnp.concatenate(
                        [prev[key][bi][:, CHUNK + delta:],
                         cur[key][bi][:, :CHUNK + delta]], axis=1))
            parts.append(ones_tile)
            pp = jnp.concatenate(parts, axis=0)
            xraw = x_ref[bi, :, pl.ds(c0, CHUNK)]
            dn = (((0,), (1,)), ((), ()))
            acc = jax.lax.dot_general(xraw, wx_ref[...], dn,
                                      preferred_element_type=jnp.float32)
            acc = acc + jax.lax.dot_general(pp, w2_ref[...], dn,
                                            preferred_element_type=jnp.float32)
            outs.append(acc)
        o_ref[0, pl.ds(c0, CHUNK), :] = outs[0].astype(o_ref.dtype)
        o_ref[1, pl.ds(c0, CHUNK), :] = outs[1].astype(o_ref.dtype)
        prev = cur


def kernel(feature_nchw, w_conv, b_conv):
    b, c, h, w = feature_nchw.shape
    k = 7
    pad = k // 2
    cout = w_conv.shape[0]
    assert w_conv.shape[1] == c + 4 * k
    hw = h * w
    shifts = _shift_offsets(k)

    xflat = feature_nchw.reshape(b, c, hw)
    wx = w_conv[:, :c]
    w2 = jnp.concatenate(
        [jnp.repeat(w_conv[:, c:], 8, axis=1),
         b_conv.reshape(cout, 1),
         jnp.zeros((cout, 7), w_conv.dtype)], axis=1)

    body = functools.partial(_escn_body, C=c, HW=hw, W=w, PAD=pad,
                             SHIFTS=shifts, CHUNK=1024)
    padw = ((pad * w + pad + 127) // 128) * 128

    out = pl.pallas_call(
        body,
        out_shape=jax.ShapeDtypeStruct((b, hw, cout), feature_nchw.dtype),
        grid=(b // 2,),
        in_specs=[
            pl.BlockSpec((2, c, hw), lambda i: (i, 0, 0)),
            pl.BlockSpec((cout, c), lambda i: (0, 0)),
            pl.BlockSpec((cout, 4 * k * 8 + 8), lambda i: (0, 0)),
        ],
        out_specs=pl.BlockSpec((2, hw, cout), lambda i: (i, 0, 0)),
        scratch_shapes=[
            pltpu.VMEM((c, hw + padw), jnp.int32),
        ],
        compiler_params=pltpu.CompilerParams(
            dimension_semantics=("parallel",),
            fuse_transposed_lhs_in_matmul=True),
    )(xflat, wx, w2)

    return out.reshape(b, h, w, cout).transpose(0, 3, 1, 2)


# speedup vs baseline: 3.8539x; 1.0581x over previous
import functools

import jax
import jax.numpy as jnp
from jax.experimental import pallas as pl
from jax.experimental.pallas import tpu as pltpu


def _shift_offsets(k):
    half = k // 2
    shifts = [(i, i) for i in range(k)]
    shifts += [(i, half) for i in range(k)]
    shifts += [(i, k - 1 - i) for i in range(k)]
    shifts += [(half, i) for i in range(k)]
    return tuple(shifts)


def _escn_body(x_ref, wx_ref, w2_ref, o_ref, pad_ref, *,
               C, HW, W, PAD, SHIFTS, CHUNK):
    groups = C // 8

    def _norm_t(bi, c0):
        xc = jnp.transpose(x_ref[bi, pl.ds(c0, CHUNK), :])
        ssq = jnp.sum(xc * xc, axis=0, keepdims=True)
        return xc * jax.lax.rsqrt(jnp.maximum(ssq, 1e-24))

    for c0 in range(0, HW, CHUNK):
        pad_ref[:, c0:c0 + CHUNK] = pltpu.pack_elementwise(
            [_norm_t(0, c0), _norm_t(1, c0)], packed_dtype=jnp.bfloat16)
    pad_ref[:, HW:] = jnp.zeros((C, pad_ref.shape[1] - HW), jnp.int32)

    pos_keys = []
    for (r, c) in SHIFTS:
        delta = (r - PAD) * W + (c - PAD)
        key = (delta, c - PAD) if delta >= 0 else (-delta, PAD - c)
        if key not in pos_keys:
            pos_keys.append(key)

    wcol = jax.lax.broadcasted_iota(jnp.int32, (1, HW), 1) & (W - 1)
    col_masks = {}
    for (_, d) in pos_keys:
        if d > 0 and d not in col_masks:
            col_masks[d] = (wcol <= (W - 1 - d)).astype(jnp.float32)
        elif d < 0 and d not in col_masks:
            col_masks[d] = (wcol >= -d).astype(jnp.float32)

    sub = jax.lax.broadcasted_iota(jnp.int32, (8, CHUNK), 0)
    ones_tile = jnp.where(sub == 0, 1.0, 0.0)

    zeros8 = jnp.zeros((8, CHUNK), jnp.float32)
    prev = {key: (zeros8, zeros8) for key in pos_keys}

    for c0 in range(0, HW, CHUNK):
        xc_bf = pltpu.bitcast(pad_ref[:, c0:c0 + CHUNK], jnp.bfloat16)
        cur = {}
        for (delta, d) in pos_keys:
            win = pad_ref[:, c0 + delta:c0 + delta + CHUNK]
            prod = pltpu.bitcast(win, jnp.bfloat16) * xc_bf
            p16 = jnp.sum(prod.reshape(groups, 16, CHUNK), axis=0)
            p16_i = pltpu.bitcast(p16, jnp.int32)
            ps = [pltpu.unpack_elementwise(
                      p16_i, index=i, packed_dtype=jnp.bfloat16,
                      unpacked_dtype=jnp.float32) for i in (0, 1)]
            if d != 0:
                m = col_masks[d][:, c0:c0 + CHUNK]
                ps = [p * m for p in ps]
            cur[(delta, d)] = ps

        for bi in range(2):
            parts = []
            for (r, c) in SHIFTS:
                delta = (r - PAD) * W + (c - PAD)
                if delta >= 0:
                    parts.append(cur[(delta, c - PAD)][bi])
                else:
                    key = (-delta, PAD - c)
                    parts.append(jnp.concatenate(
                        [prev[key][bi][:, CHUNK + delta:],
                         cur[key][bi][:, :CHUNK + delta]], axis=1))
            parts.append(ones_tile)
            pp = jnp.concatenate(parts, axis=0)
            xraw = x_ref[bi, pl.ds(c0, CHUNK), :]
            acc = jnp.dot(xraw, wx_ref[...], preferred_element_type=jnp.float32)
            acc = acc + jax.lax.dot_general(
                pp, w2_ref[...], (((0,), (1,)), ((), ())),
                preferred_element_type=jnp.float32)
            o_ref[bi, pl.ds(c0, CHUNK), :] = acc.astype(o_ref.dtype)
        prev = cur


def kernel(feature_nchw, w_conv, b_conv):
    b, c, h, w = feature_nchw.shape
    k = 7
    pad = k // 2
    cout = w_conv.shape[0]
    assert w_conv.shape[1] == c + 4 * k
    hw = h * w
    shifts = _shift_offsets(k)

    x_nhwc = feature_nchw.transpose(0, 2, 3, 1).reshape(b, hw, c)
    wx = jnp.transpose(w_conv[:, :c])
    w2 = jnp.concatenate(
        [jnp.repeat(w_conv[:, c:], 8, axis=1),
         b_conv.reshape(cout, 1),
         jnp.zeros((cout, 7), w_conv.dtype)], axis=1)

    body = functools.partial(_escn_body, C=c, HW=hw, W=w, PAD=pad,
                             SHIFTS=shifts, CHUNK=1024)
    padw = ((pad * w + pad + 127) // 128) * 128

    out = pl.pallas_call(
        body,
        out_shape=jax.ShapeDtypeStruct((b, hw, cout), feature_nchw.dtype),
        grid=(b // 2,),
        in_specs=[
            pl.BlockSpec((2, hw, c), lambda i: (i, 0, 0)),
            pl.BlockSpec((c, cout), lambda i: (0, 0)),
            pl.BlockSpec((cout, 4 * k * 8 + 8), lambda i: (0, 0)),
        ],
        out_specs=pl.BlockSpec((2, hw, cout), lambda i: (i, 0, 0)),
        scratch_shapes=[
            pltpu.VMEM((c, hw + padw), jnp.int32),
        ],
        compiler_params=pltpu.CompilerParams(
            dimension_semantics=("parallel",),
            fuse_transposed_lhs_in_matmul=True),
    )(x_nhwc, wx, w2)

    return out.reshape(b, h, w, cout).transpose(0, 3, 1, 2)


# speedup vs baseline: 4.0067x; 1.0397x over previous
import functools

import jax
import jax.numpy as jnp
from jax.experimental import pallas as pl
from jax.experimental.pallas import tpu as pltpu


def _shift_offsets(k):
    half = k // 2
    shifts = [(i, i) for i in range(k)]
    shifts += [(i, half) for i in range(k)]
    shifts += [(i, k - 1 - i) for i in range(k)]
    shifts += [(half, i) for i in range(k)]
    return tuple(shifts)


def _escn_body(x_ref, wx_ref, w2_ref, o_ref, pad_ref, *,
               C, HW, W, PAD, SHIFTS, CHUNK):
    groups = C // 8

    def _norm_t(bi, c0):
        xc = jnp.transpose(x_ref[bi, pl.ds(c0, CHUNK), :])
        ssq = jnp.sum(xc * xc, axis=0, keepdims=True)
        return xc * jax.lax.rsqrt(jnp.maximum(ssq, 1e-24))

    for c0 in range(0, HW, CHUNK):
        pad_ref[:, c0:c0 + CHUNK] = pltpu.pack_elementwise(
            [_norm_t(0, c0), _norm_t(1, c0)], packed_dtype=jnp.bfloat16)
    pad_ref[:, HW:] = jnp.zeros((C, pad_ref.shape[1] - HW), jnp.int32)

    pos_keys = []
    for (r, c) in SHIFTS:
        delta = (r - PAD) * W + (c - PAD)
        key = (delta, c - PAD) if delta >= 0 else (-delta, PAD - c)
        if key not in pos_keys:
            pos_keys.append(key)

    wcol = jax.lax.broadcasted_iota(jnp.int32, (1, HW), 1) & (W - 1)
    col_masks = {}
    for (_, d) in pos_keys:
        if d > 0 and d not in col_masks:
            col_masks[d] = (wcol <= (W - 1 - d)).astype(jnp.float32)
        elif d < 0 and d not in col_masks:
            col_masks[d] = (wcol >= -d).astype(jnp.float32)

    sub = jax.lax.broadcasted_iota(jnp.int32, (8, CHUNK), 0)
    ones_tile = jnp.where(sub == 0, 1.0, 0.0)

    zeros8 = jnp.zeros((8, CHUNK), jnp.float32)
    prev = {key: (zeros8, zeros8) for key in pos_keys}

    for c0 in range(0, HW, CHUNK):
        xc_bf = pltpu.bitcast(pad_ref[:, c0:c0 + CHUNK], jnp.bfloat16)
        cur = {}
        for (delta, d) in pos_keys:
            win = pad_ref[:, c0 + delta:c0 + delta + CHUNK]
            prod = pltpu.bitcast(win, jnp.bfloat16) * xc_bf
            terms = [prod[16 * g:16 * g + 16, :] for g in range(groups)]
            while len(terms) > 1:
                terms = [terms[i] + terms[i + 1]
                         for i in range(0, len(terms), 2)]
            p16_i = pltpu.bitcast(terms[0], jnp.int32)
            ps = [pltpu.unpack_elementwise(
                      p16_i, index=i, packed_dtype=jnp.bfloat16,
                      unpacked_dtype=jnp.float32) for i in (0, 1)]
            if d != 0:
                m = col_masks[d][:, c0:c0 + CHUNK]
                ps = [p * m for p in ps]
            cur[(delta, d)] = ps

        for bi in range(2):
            parts = []
            for (r, c) in SHIFTS:
                delta = (r - PAD) * W + (c - PAD)
                if delta >= 0:
                    parts.append(cur[(delta, c - PAD)][bi])
                else:
                    key = (-delta, PAD - c)
                    parts.append(jnp.concatenate(
                        [prev[key][bi][:, CHUNK + delta:],
                         cur[key][bi][:, :CHUNK + delta]], axis=1))
            parts.append(ones_tile)
            pp = jnp.concatenate(parts, axis=0)
            xraw = x_ref[bi, pl.ds(c0, CHUNK), :]
            acc = jnp.dot(xraw, wx_ref[...], preferred_element_type=jnp.float32)
            acc = acc + jax.lax.dot_general(
                pp, w2_ref[...], (((0,), (1,)), ((), ())),
                preferred_element_type=jnp.float32)
            o_ref[bi, pl.ds(c0, CHUNK), :] = acc.astype(o_ref.dtype)
        prev = cur


def kernel(feature_nchw, w_conv, b_conv):
    b, c, h, w = feature_nchw.shape
    k = 7
    pad = k // 2
    cout = w_conv.shape[0]
    assert w_conv.shape[1] == c + 4 * k
    hw = h * w
    shifts = _shift_offsets(k)

    x_nhwc = feature_nchw.transpose(0, 2, 3, 1).reshape(b, hw, c)
    wx = jnp.transpose(w_conv[:, :c])
    w2 = jnp.concatenate(
        [jnp.repeat(w_conv[:, c:], 8, axis=1),
         b_conv.reshape(cout, 1),
         jnp.zeros((cout, 7), w_conv.dtype)], axis=1)

    body = functools.partial(_escn_body, C=c, HW=hw, W=w, PAD=pad,
                             SHIFTS=shifts, CHUNK=1024)
    padw = ((pad * w + pad + 127) // 128) * 128

    out = pl.pallas_call(
        body,
        out_shape=jax.ShapeDtypeStruct((b, hw, cout), feature_nchw.dtype),
        grid=(b // 2,),
        in_specs=[
            pl.BlockSpec((2, hw, c), lambda i: (i, 0, 0)),
            pl.BlockSpec((c, cout), lambda i: (0, 0)),
            pl.BlockSpec((cout, 4 * k * 8 + 8), lambda i: (0, 0)),
        ],
        out_specs=pl.BlockSpec((2, hw, cout), lambda i: (i, 0, 0)),
        scratch_shapes=[
            pltpu.VMEM((c, hw + padw), jnp.int32),
        ],
        compiler_params=pltpu.CompilerParams(
            dimension_semantics=("parallel",),
            fuse_transposed_lhs_in_matmul=True),
    )(x_nhwc, wx, w2)

    return out.reshape(b, h, w, cout).transpose(0, 3, 1, 2)
